# baseline (speedup 1.0000x reference)
import functools

import jax
import jax.numpy as jnp
from jax.experimental import pallas as pl
from jax.experimental.pallas import tpu as pltpu

N_LEFT = 8
N_RIGHT = 8
N_HEADS = 12
_BT = 128
_KW = _BT + 16
_NEG = -1e9
_CSHIFT = -30.0


def _attn_body(e_ref, isn_ref, w3_ref, wo_ref, o_ref, *, T, D):
    H = N_HEADS
    dh = D // H
    i = pl.program_id(1)
    t0 = i * _BT
    h0 = jnp.maximum(0, jnp.minimum(t0 - N_LEFT, T - _KW))
    h0 = pl.multiple_of(h0, 8)
    delta = t0 - h0

    ec = e_ref[0, pl.ds(t0, _BT), :]
    eh = e_ref[0, pl.ds(h0, _KW), :].astype(jnp.bfloat16)
    q = jnp.dot(ec.astype(jnp.bfloat16), w3_ref[:, 0:D],
                preferred_element_type=jnp.float32).astype(jnp.bfloat16)
    kv = jnp.dot(eh, w3_ref[:, D:3 * D],
                 preferred_element_type=jnp.float32).astype(jnp.bfloat16)
    k = kv[:, 0:D]
    v = kv[:, D:2 * D]

    qi = jax.lax.broadcasted_iota(jnp.int32, (_BT, _KW), 0)
    kj = jax.lax.broadcasted_iota(jnp.int32, (_BT, _KW), 1)
    rel = kj - qi - delta
    mask = (rel >= -N_LEFT) & (rel <= N_RIGHT)
    bias = jnp.where(mask, jnp.float32(_CSHIFT), jnp.float32(_NEG))
    ones = jnp.ones((_KW, 1), dtype=jnp.bfloat16)

    outs = []
    for h in range(H):
        sl = slice(h * dh, (h + 1) * dh)
        s = jax.lax.dot_general(q[:, sl], k[:, sl],
                                (((1,), (1,)), ((), ())),
                                preferred_element_type=jnp.float32)
        p = jnp.exp(s + bias).astype(jnp.bfloat16)
        ssum = jnp.dot(p, ones, preferred_element_type=jnp.float32)
        o = jnp.dot(p, v[:, sl], preferred_element_type=jnp.float32)
        outs.append(o / ssum)
    attn = jnp.concatenate(outs, axis=1).astype(jnp.bfloat16)
    a = jnp.dot(attn, wo_ref[...], preferred_element_type=jnp.float32)

    msk = isn_ref[0] != 0
    o_ref[0] = jnp.where(msk, a, ec)


def kernel(embeds, is_numbers, Wq, Wk, Wv, Wo):
    B, T, D = embeds.shape
    dh = D // N_HEADS
    isn = is_numbers.astype(jnp.int32).reshape(B, T, 1)
    scale = 1.0 / (dh ** 0.5)
    w3 = jnp.concatenate([Wq * scale, Wk, Wv], axis=1).astype(jnp.bfloat16)
    wo = Wo.astype(jnp.bfloat16)
    return pl.pallas_call(
        functools.partial(_attn_body, T=T, D=D),
        grid=(B, T // _BT),
        in_specs=[
            pl.BlockSpec((1, T, D), lambda b, i: (b, 0, 0)),
            pl.BlockSpec((1, _BT, 1), lambda b, i: (b, i, 0)),
            pl.BlockSpec((D, 3 * D), lambda b, i: (0, 0)),
            pl.BlockSpec((D, D), lambda b, i: (0, 0)),
        ],
        out_specs=pl.BlockSpec((1, _BT, D), lambda b, i: (b, i, 0)),
        out_shape=jax.ShapeDtypeStruct((B, T, D), jnp.float32),
        compiler_params=pltpu.CompilerParams(
            dimension_semantics=("parallel", "arbitrary"),
        ),
    )(embeds, isn, w3, wo)


# speedup vs baseline: 1.2481x; 1.2481x over previous
import functools

import jax
import jax.numpy as jnp
from jax.experimental import pallas as pl
from jax.experimental.pallas import tpu as pltpu

N_LEFT = 8
N_RIGHT = 8
N_HEADS = 12
_BT = 128
_KW = _BT + 16
_NEG = -1e9
_CSHIFT = -30.0


def _attn_body(e_ref, isn_ref, w3_ref, wo_ref, o_ref, *, B, T, D):
    H = N_HEADS
    dh = D // H
    i = pl.program_id(0)
    t0 = i * _BT
    h0 = jnp.maximum(0, jnp.minimum(t0 - N_LEFT, T - _KW))
    h0 = pl.multiple_of(h0, 8)
    delta = t0 - h0

    qi = jax.lax.broadcasted_iota(jnp.int32, (_BT, _KW), 0)
    kj = jax.lax.broadcasted_iota(jnp.int32, (_BT, _KW), 1)
    rel = kj - qi - delta
    mask = (rel >= -N_LEFT) & (rel <= N_RIGHT)
    bias = jnp.where(mask, jnp.float32(_CSHIFT), jnp.float32(_NEG))

    for b in range(B):
        ec = e_ref[b, pl.ds(t0, _BT), :]
        eh = e_ref[b, pl.ds(h0, _KW), :].astype(jnp.bfloat16)
        q = jnp.dot(ec.astype(jnp.bfloat16), w3_ref[:, 0:D],
                    preferred_element_type=jnp.float32).astype(jnp.bfloat16)
        kv = jnp.dot(eh, w3_ref[:, D:3 * D],
                     preferred_element_type=jnp.float32).astype(jnp.bfloat16)
        k = kv[:, 0:D]
        v = kv[:, D:2 * D]

        outs = []
        for h in range(H):
            sl = slice(h * dh, (h + 1) * dh)
            s = jax.lax.dot_general(q[:, sl], k[:, sl],
                                    (((1,), (1,)), ((), ())),
                                    preferred_element_type=jnp.float32)
            p = jnp.exp(s + bias)
            r = 1.0 / jnp.sum(p, axis=1, keepdims=True)
            o = jnp.dot(p.astype(jnp.bfloat16), v[:, sl],
                        preferred_element_type=jnp.float32)
            outs.append(o * r)
        attn = jnp.concatenate(outs, axis=1).astype(jnp.bfloat16)
        a = jnp.dot(attn, wo_ref[...], preferred_element_type=jnp.float32)

        msk = isn_ref[b] != 0
        o_ref[b] = jnp.where(msk, a, ec)


def kernel(embeds, is_numbers, Wq, Wk, Wv, Wo):
    B, T, D = embeds.shape
    dh = D // N_HEADS
    isn = is_numbers.astype(jnp.int32).reshape(B, T, 1)
    scale = 1.0 / (dh ** 0.5)
    w3 = jnp.concatenate([Wq * scale, Wk, Wv], axis=1).astype(jnp.bfloat16)
    wo = Wo.astype(jnp.bfloat16)
    return pl.pallas_call(
        functools.partial(_attn_body, B=B, T=T, D=D),
        grid=(T // _BT,),
        in_specs=[
            pl.BlockSpec((B, T, D), lambda i: (0, 0, 0)),
            pl.BlockSpec((B, _BT, 1), lambda i: (0, i, 0)),
            pl.BlockSpec((D, 3 * D), lambda i: (0, 0)),
            pl.BlockSpec((D, D), lambda i: (0, 0)),
        ],
        out_specs=pl.BlockSpec((B, _BT, D), lambda i: (0, i, 0)),
        out_shape=jax.ShapeDtypeStruct((B, T, D), jnp.float32),
        compiler_params=pltpu.CompilerParams(
            dimension_semantics=("arbitrary",),
        ),
    )(embeds, isn, w3, wo)


# speedup vs baseline: 1.3652x; 1.0939x over previous
import functools

import jax
import jax.numpy as jnp
from jax.experimental import pallas as pl
from jax.experimental.pallas import tpu as pltpu

N_LEFT = 8
N_RIGHT = 8
N_HEADS = 12
_BT = 128
_KW = _BT + 16
_NEG = -1e9
_CSHIFT = -30.0


def _attn_body(e_ref, isn_ref, w3_ref, wo_ref, o_ref, *, T, D):
    H = N_HEADS
    dh = D // H
    i = pl.program_id(1)
    t0 = i * _BT
    h0 = jnp.maximum(0, jnp.minimum(t0 - N_LEFT, T - _KW))
    h0 = pl.multiple_of(h0, 8)
    delta = t0 - h0

    ec = e_ref[0, pl.ds(t0, _BT), :]
    eh = e_ref[0, pl.ds(h0, _KW), :].astype(jnp.bfloat16)
    q = jnp.dot(ec.astype(jnp.bfloat16), w3_ref[:, 0:D],
                preferred_element_type=jnp.float32).astype(jnp.bfloat16)
    kv = jnp.dot(eh, w3_ref[:, D:3 * D],
                 preferred_element_type=jnp.float32).astype(jnp.bfloat16)
    k = kv[:, 0:D]
    v = kv[:, D:2 * D]

    qi = jax.lax.broadcasted_iota(jnp.int32, (_BT, _KW), 0)
    kj = jax.lax.broadcasted_iota(jnp.int32, (_BT, _KW), 1)
    rel = kj - qi - delta
    mask = (rel >= -N_LEFT) & (rel <= N_RIGHT)
    bias = jnp.where(mask, jnp.float32(_CSHIFT), jnp.float32(_NEG))

    outs = []
    for h in range(H):
        sl = slice(h * dh, (h + 1) * dh)
        s = jax.lax.dot_general(q[:, sl], k[:, sl],
                                (((1,), (1,)), ((), ())),
                                preferred_element_type=jnp.float32)
        p = jnp.exp(s + bias)
        r = 1.0 / jnp.sum(p, axis=1, keepdims=True)
        o = jnp.dot(p.astype(jnp.bfloat16), v[:, sl],
                    preferred_element_type=jnp.float32)
        outs.append(o * r)
    attn = jnp.concatenate(outs, axis=1).astype(jnp.bfloat16)
    a = jnp.dot(attn, wo_ref[...], preferred_element_type=jnp.float32)

    msk = isn_ref[0, pl.ds(t0, _BT), :] != 0
    o_ref[0] = jnp.where(msk, a, ec)


def kernel(embeds, is_numbers, Wq, Wk, Wv, Wo):
    B, T, D = embeds.shape
    dh = D // N_HEADS
    isn = is_numbers.astype(jnp.int32).reshape(B, T, 1)
    scale = 1.0 / (dh ** 0.5)
    w3 = jnp.concatenate([Wq * scale, Wk, Wv], axis=1).astype(jnp.bfloat16)
    wo = Wo.astype(jnp.bfloat16)
    return pl.pallas_call(
        functools.partial(_attn_body, T=T, D=D),
        grid=(B, T // _BT),
        in_specs=[
            pl.BlockSpec((1, T, D), lambda b, i: (b, 0, 0)),
            pl.BlockSpec((1, T, 1), lambda b, i: (b, 0, 0)),
            pl.BlockSpec((D, 3 * D), lambda b, i: (0, 0)),
            pl.BlockSpec((D, D), lambda b, i: (0, 0)),
        ],
        out_specs=pl.BlockSpec((1, _BT, D), lambda b, i: (b, i, 0)),
        out_shape=jax.ShapeDtypeStruct((B, T, D), jnp.float32),
        compiler_params=pltpu.CompilerParams(
            dimension_semantics=("parallel", "arbitrary"),
        ),
    )(embeds, isn, w3, wo)


# speedup vs baseline: 1.3891x; 1.0175x over previous
import functools

import jax
import jax.numpy as jnp
from jax.experimental import pallas as pl
from jax.experimental.pallas import tpu as pltpu

N_LEFT = 8
N_RIGHT = 8
N_HEADS = 12
_BT = 128
_KW = _BT + 16
_NEG = -1e9
_CSHIFT = -30.0


def _attn_body(prev_ref, cur_ref, nxt_ref, isn_ref, w3_ref, wo_ref, o_ref,
               *, T, D):
    H = N_HEADS
    dh = D // H
    i = pl.program_id(1)
    t0 = i * _BT

    ec = cur_ref[0]
    ecb = ec.astype(jnp.bfloat16)
    left = prev_ref[0, _BT - N_LEFT:_BT, :].astype(jnp.bfloat16)
    right = nxt_ref[0, 0:N_RIGHT, :].astype(jnp.bfloat16)
    eh = jnp.concatenate([left, ecb, right], axis=0)
    q = jnp.dot(ecb, w3_ref[:, 0:D],
                preferred_element_type=jnp.float32).astype(jnp.bfloat16)
    kv = jnp.dot(eh, w3_ref[:, D:3 * D],
                 preferred_element_type=jnp.float32).astype(jnp.bfloat16)
    k = kv[:, 0:D]
    v = kv[:, D:2 * D]

    qi = jax.lax.broadcasted_iota(jnp.int32, (_BT, _KW), 0)
    kj = jax.lax.broadcasted_iota(jnp.int32, (_BT, _KW), 1)
    rel = kj - qi - N_LEFT
    pos_k = t0 + kj - N_LEFT
    mask = (rel >= -N_LEFT) & (rel <= N_RIGHT) & (pos_k >= 0) & (pos_k < T)
    bias = jnp.where(mask, jnp.float32(_CSHIFT), jnp.float32(_NEG))

    outs = []
    for h in range(H):
        sl = slice(h * dh, (h + 1) * dh)
        s = jax.lax.dot_general(q[:, sl], k[:, sl],
                                (((1,), (1,)), ((), ())),
                                preferred_element_type=jnp.float32)
        p = jnp.exp(s + bias)
        r = 1.0 / jnp.sum(p, axis=1, keepdims=True)
        o = jnp.dot(p.astype(jnp.bfloat16), v[:, sl],
                    preferred_element_type=jnp.float32)
        outs.append(o * r)
    attn = jnp.concatenate(outs, axis=1).astype(jnp.bfloat16)
    a = jnp.dot(attn, wo_ref[...], preferred_element_type=jnp.float32)

    msk = isn_ref[0, pl.ds(t0, _BT), :] != 0
    o_ref[0] = jnp.where(msk, a, ec)


def kernel(embeds, is_numbers, Wq, Wk, Wv, Wo):
    B, T, D = embeds.shape
    dh = D // N_HEADS
    nblk = T // _BT
    isn = is_numbers.astype(jnp.int32).reshape(B, T, 1)
    scale = 1.0 / (dh ** 0.5)
    w3 = jnp.concatenate([Wq * scale, Wk, Wv], axis=1).astype(jnp.bfloat16)
    wo = Wo.astype(jnp.bfloat16)
    last = nblk - 1
    return pl.pallas_call(
        functools.partial(_attn_body, T=T, D=D),
        grid=(B, nblk),
        in_specs=[
            pl.BlockSpec((1, _BT, D),
                         lambda b, i: (b, jnp.maximum(i - 1, 0), 0)),
            pl.BlockSpec((1, _BT, D), lambda b, i: (b, i, 0)),
            pl.BlockSpec((1, _BT, D),
                         lambda b, i: (b, jnp.minimum(i + 1, last), 0)),
            pl.BlockSpec((1, T, 1), lambda b, i: (b, 0, 0)),
            pl.BlockSpec((D, 3 * D), lambda b, i: (0, 0)),
            pl.BlockSpec((D, D), lambda b, i: (0, 0)),
        ],
        out_specs=pl.BlockSpec((1, _BT, D), lambda b, i: (b, i, 0)),
        out_shape=jax.ShapeDtypeStruct((B, T, D), jnp.float32),
        compiler_params=pltpu.CompilerParams(
            dimension_semantics=("arbitrary", "arbitrary"),
        ),
    )(embeds, embeds, embeds, isn, w3, wo)


# speedup vs baseline: 1.4634x; 1.0536x over previous
import functools

import jax
import jax.numpy as jnp
from jax.experimental import pallas as pl
from jax.experimental.pallas import tpu as pltpu

N_LEFT = 8
N_RIGHT = 8
N_HEADS = 12
_BT = 128
_KW = _BT + 16
_NEG = -1e9
_CSHIFT = -30.0


def _attn_body(prev_ref, cur_ref, nxt_ref, isn_ref,
               wq_ref, wk_ref, wv_ref, wo_ref, o_ref, w3s_ref, wos_ref,
               *, T, D):
    H = N_HEADS
    dh = D // H
    i = pl.program_id(1)
    t0 = i * _BT

    @pl.when(jnp.logical_and(pl.program_id(0) == 0, i == 0))
    def _cast_weights():
        scale = jnp.float32(1.0 / (dh ** 0.5))
        w3s_ref[:, 0:D] = (wq_ref[...] * scale).astype(jnp.bfloat16)
        w3s_ref[:, D:2 * D] = wk_ref[...].astype(jnp.bfloat16)
        w3s_ref[:, 2 * D:3 * D] = wv_ref[...].astype(jnp.bfloat16)
        wos_ref[...] = wo_ref[...].astype(jnp.bfloat16)

    ec = cur_ref[0]
    ecb = ec.astype(jnp.bfloat16)
    left = prev_ref[0, _BT - N_LEFT:_BT, :].astype(jnp.bfloat16)
    right = nxt_ref[0, 0:N_RIGHT, :].astype(jnp.bfloat16)
    eh = jnp.concatenate([left, ecb, right], axis=0)
    q = jnp.dot(ecb, w3s_ref[:, 0:D],
                preferred_element_type=jnp.float32).astype(jnp.bfloat16)
    kv = jnp.dot(eh, w3s_ref[:, D:3 * D],
                 preferred_element_type=jnp.float32).astype(jnp.bfloat16)
    k = kv[:, 0:D]
    v = kv[:, D:2 * D]

    qi = jax.lax.broadcasted_iota(jnp.int32, (_BT, _KW), 0)
    kj = jax.lax.broadcasted_iota(jnp.int32, (_BT, _KW), 1)
    rel = kj - qi - N_LEFT
    pos_k = t0 + kj - N_LEFT
    mask = (rel >= -N_LEFT) & (rel <= N_RIGHT) & (pos_k >= 0) & (pos_k < T)
    bias = jnp.where(mask, jnp.float32(_CSHIFT), jnp.float32(_NEG))

    outs = []
    for h in range(H):
        sl = slice(h * dh, (h + 1) * dh)
        s = jax.lax.dot_general(q[:, sl], k[:, sl],
                                (((1,), (1,)), ((), ())),
                                preferred_element_type=jnp.float32)
        p = jnp.exp(s + bias)
        r = 1.0 / jnp.sum(p, axis=1, keepdims=True)
        o = jnp.dot(p.astype(jnp.bfloat16), v[:, sl],
                    preferred_element_type=jnp.float32)
        outs.append(o * r)
    attn = jnp.concatenate(outs, axis=1).astype(jnp.bfloat16)
    a = jnp.dot(attn, wos_ref[...], preferred_element_type=jnp.float32)

    msk = isn_ref[0, pl.ds(t0, _BT), :] != 0
    o_ref[0] = jnp.where(msk, a, ec)


def kernel(embeds, is_numbers, Wq, Wk, Wv, Wo):
    B, T, D = embeds.shape
    nblk = T // _BT
    isn = is_numbers.astype(jnp.int32).reshape(B, T, 1)
    last = nblk - 1
    return pl.pallas_call(
        functools.partial(_attn_body, T=T, D=D),
        grid=(B, nblk),
        in_specs=[
            pl.BlockSpec((1, _BT, D),
                         lambda b, i: (b, jnp.maximum(i - 1, 0), 0)),
            pl.BlockSpec((1, _BT, D), lambda b, i: (b, i, 0)),
            pl.BlockSpec((1, _BT, D),
                         lambda b, i: (b, jnp.minimum(i + 1, last), 0)),
            pl.BlockSpec((1, T, 1), lambda b, i: (b, 0, 0)),
            pl.BlockSpec((D, D), lambda b, i: (0, 0)),
            pl.BlockSpec((D, D), lambda b, i: (0, 0)),
            pl.BlockSpec((D, D), lambda b, i: (0, 0)),
            pl.BlockSpec((D, D), lambda b, i: (0, 0)),
        ],
        out_specs=pl.BlockSpec((1, _BT, D), lambda b, i: (b, i, 0)),
        out_shape=jax.ShapeDtypeStruct((B, T, D), jnp.float32),
        scratch_shapes=[
            pltpu.VMEM((D, 3 * D), jnp.bfloat16),
            pltpu.VMEM((D, D), jnp.bfloat16),
        ],
        compiler_params=pltpu.CompilerParams(
            dimension_semantics=("arbitrary", "arbitrary"),
        ),
    )(embeds, embeds, embeds, isn, Wq, Wk, Wv, Wo)


# speedup vs baseline: 1.4756x; 1.0083x over previous
import functools

import jax
import jax.numpy as jnp
from jax.experimental import pallas as pl
from jax.experimental.pallas import tpu as pltpu

N_LEFT = 8
N_RIGHT = 8
N_HEADS = 12
_BT = 128
_KW = _BT + 16
_NEG = -1e9
_CSHIFT = -30.0


def _attn_body(prev_ref, cur_ref, nxt_ref, isn_ref,
               wq_ref, wk_ref, wv_ref, wo_ref, o_ref, w3s_ref, wos_ref,
               *, T, D):
    H = N_HEADS
    dh = D // H
    i = pl.program_id(1)
    t0 = i * _BT

    @pl.when(jnp.logical_and(pl.program_id(0) == 0, i == 0))
    def _cast_weights():
        scale = jnp.float32(1.0 / (dh ** 0.5))
        w3s_ref[:, 0:D] = (wq_ref[...] * scale).astype(jnp.bfloat16)
        w3s_ref[:, D:2 * D] = wk_ref[...].astype(jnp.bfloat16)
        w3s_ref[:, 2 * D:3 * D] = wv_ref[...].astype(jnp.bfloat16)
        wos_ref[...] = wo_ref[...].astype(jnp.bfloat16)

    ec = cur_ref[0]
    ecb = ec.astype(jnp.bfloat16)
    left = prev_ref[0, _BT - N_LEFT:_BT, :].astype(jnp.bfloat16)
    right = nxt_ref[0, 0:N_RIGHT, :].astype(jnp.bfloat16)
    eh = jnp.concatenate([left, ecb, right], axis=0)
    q = jnp.dot(ecb, w3s_ref[:, 0:D],
                preferred_element_type=jnp.float32).astype(jnp.bfloat16)
    kv = jnp.dot(eh, w3s_ref[:, D:3 * D],
                 preferred_element_type=jnp.float32).astype(jnp.bfloat16)
    k = kv[:, 0:D]
    v = kv[:, D:2 * D]

    qi = jax.lax.broadcasted_iota(jnp.int32, (_BT, _KW), 0)
    kj = jax.lax.broadcasted_iota(jnp.int32, (_BT, _KW), 1)
    rel = kj - qi - N_LEFT
    pos_k = t0 + kj - N_LEFT
    mask = (rel >= -N_LEFT) & (rel <= N_RIGHT) & (pos_k >= 0) & (pos_k < T)
    bias = jnp.where(mask, jnp.float32(_CSHIFT), jnp.float32(_NEG))

    outs = []
    for h in range(H):
        sl = slice(h * dh, (h + 1) * dh)
        s = jax.lax.dot_general(q[:, sl], k[:, sl],
                                (((1,), (1,)), ((), ())),
                                preferred_element_type=jnp.float32)
        p = jnp.exp(s + bias)
        r = 1.0 / jnp.sum(p, axis=1, keepdims=True)
        o = jnp.dot(p.astype(jnp.bfloat16), v[:, sl],
                    preferred_element_type=jnp.float32)
        outs.append((o * r).astype(jnp.bfloat16))
    attn = jnp.concatenate(outs, axis=1)
    a = jnp.dot(attn, wos_ref[...], preferred_element_type=jnp.float32)

    msk = isn_ref[0, pl.ds(t0, _BT), :]
    o_ref[0] = jnp.where(msk, a, ec)


def kernel(embeds, is_numbers, Wq, Wk, Wv, Wo):
    B, T, D = embeds.shape
    nblk = T // _BT
    isn = is_numbers.reshape(B, T, 1)
    last = nblk - 1
    return pl.pallas_call(
        functools.partial(_attn_body, T=T, D=D),
        grid=(B, nblk),
        in_specs=[
            pl.BlockSpec((1, _BT, D),
                         lambda b, i: (b, jnp.maximum(i - 1, 0), 0)),
            pl.BlockSpec((1, _BT, D), lambda b, i: (b, i, 0)),
            pl.BlockSpec((1, _BT, D),
                         lambda b, i: (b, jnp.minimum(i + 1, last), 0)),
            pl.BlockSpec((1, T, 1), lambda b, i: (b, 0, 0)),
            pl.BlockSpec((D, D), lambda b, i: (0, 0)),
            pl.BlockSpec((D, D), lambda b, i: (0, 0)),
            pl.BlockSpec((D, D), lambda b, i: (0, 0)),
            pl.BlockSpec((D, D), lambda b, i: (0, 0)),
        ],
        out_specs=pl.BlockSpec((1, _BT, D), lambda b, i: (b, i, 0)),
        out_shape=jax.ShapeDtypeStruct((B, T, D), jnp.float32),
        scratch_shapes=[
            pltpu.VMEM((D, 3 * D), jnp.bfloat16),
            pltpu.VMEM((D, D), jnp.bfloat16),
        ],
        compiler_params=pltpu.CompilerParams(
            dimension_semantics=("arbitrary", "arbitrary"),
        ),
    )(embeds, embeds, embeds, isn, Wq, Wk, Wv, Wo)
